# baseline (speedup 1.0000x reference)
import functools

import jax
import jax.numpy as jnp
from jax.experimental import pallas as pl
from jax.experimental.pallas import tpu as pltpu

ALPHA_C = 3.0
K_C = 30
_BIG = 0x7FFFFFFF
_LANE = 128


def _bisect_while(count_ge, k, lo0, hi0):
    def cond(carry):
        lo, hi = carry
        return jnp.max(hi - lo) > 1

    def step(carry):
        lo, hi = carry
        mid = lo + ((hi - lo) >> 1)
        ok = count_ge(mid) >= k
        return (jnp.where(ok, mid, lo), jnp.where(ok, hi, mid))

    def body(carry):
        return step(step(carry))

    lo, _ = jax.lax.while_loop(cond, body, (lo0, hi0))
    return lo


def _body(a_ref, o_ref, u_ref, *, n_cols, k):
    x = a_ref[...]
    adj = jnp.tanh(ALPHA_C * x)
    o_ref[...] = adj
    u = jax.lax.bitcast_convert_type(jnp.abs(adj), jnp.int32)
    br = x.shape[0]
    n_pad = u_ref.shape[1]
    u_ref[:, n_cols:n_pad] = jnp.zeros((br, n_pad - n_cols), jnp.int32)
    u_ref[:, 0:n_cols] = u

    w_mx = _LANE
    mx = u_ref[:, 0:w_mx]
    for i in range(1, n_pad // w_mx):
        mx = jnp.maximum(mx, u_ref[:, i * w_mx:(i + 1) * w_mx])
    rowmax = jnp.max(mx, axis=1, keepdims=True)
    rowmin = jnp.min(mx, axis=1, keepdims=True)

    def count_mx(t):
        return jnp.sum((mx >= t).astype(jnp.int32), axis=1, keepdims=True)

    m_k = _bisect_while(count_mx, k, rowmin, rowmax + 1)

    def count_u(t):
        return jnp.sum((u_ref[...] >= t).astype(jnp.int32), axis=1,
                       keepdims=True)

    t_bits = _bisect_while(count_u, k, m_k, rowmax + 1)

    uu = u_ref[...]
    iota = jax.lax.broadcasted_iota(jnp.int32, (br, n_pad), 1)
    u_ref[...] = jnp.where(uu > t_bits, -1,
                           jnp.where(uu == t_bits, iota, _BIG))

    def count_le(t):
        return jnp.sum((u_ref[...] <= t).astype(jnp.int32), axis=1,
                       keepdims=True)

    lo0j = jnp.full((br, 1), -1, jnp.int32)
    hi0j = jnp.full((br, 1), n_cols - 1, jnp.int32)
    clo0 = count_le(lo0j)
    chi0 = count_le(hi0j)

    def jprobe(carry, mid):
        lo, c_lo, hi, c_hi = carry
        c = count_le(mid)
        ok = c >= k
        return (jnp.where(ok, lo, mid), jnp.where(ok, c_lo, c),
                jnp.where(ok, mid, hi), jnp.where(ok, c, c_hi))

    def jcond(carry):
        lo, _, hi, _ = carry
        return jnp.max(hi - lo) > 1

    def jbody(carry):
        lo, c_lo, hi, c_hi = carry
        span = hi - lo
        frac = ((k - c_lo).astype(jnp.float32) * span.astype(jnp.float32)
                / (c_hi - c_lo).astype(jnp.float32))
        step = jnp.clip(frac.astype(jnp.int32), 1, span - 1)
        carry = jprobe(carry, lo + step)
        lo, c_lo, hi, c_hi = carry
        mid2 = lo + jnp.maximum((hi - lo) >> 1, 1)
        return jprobe(carry, mid2)

    _, _, j_cut, _ = jax.lax.while_loop(jcond, jbody,
                                        (lo0j, clo0, hi0j, chi0))

    keep = u_ref[:, 0:n_cols] <= j_cut
    o_ref[...] = jnp.where(keep, o_ref[...], 0.0)


def kernel(idx, A):
    n, n_cols = A.shape
    del idx
    br = next(b for b in (200, 80, 40, 16, 8, 1) if n % b == 0)
    n_pad = ((n_cols + _LANE - 1) // _LANE) * _LANE
    if n_pad % 256:
        n_pad += _LANE
    body = functools.partial(_body, n_cols=n_cols, k=K_C)
    return pl.pallas_call(
        body,
        grid=(n // br,),
        in_specs=[pl.BlockSpec((br, n_cols), lambda i: (i, 0))],
        out_specs=pl.BlockSpec((br, n_cols), lambda i: (i, 0)),
        out_shape=jax.ShapeDtypeStruct((n, n_cols), jnp.float32),
        scratch_shapes=[
            pltpu.VMEM((br, n_pad), jnp.int32),
        ],
    )(A)


# speedup vs baseline: 1.6256x; 1.6256x over previous
import functools

import jax
import jax.numpy as jnp
from jax.experimental import pallas as pl
from jax.experimental.pallas import tpu as pltpu

ALPHA_C = 3.0
K_C = 30
_BIG = 0x7FFFFFFF
_LANE = 128


def _bisect_while(count_ge, k, lo0, hi0):
    def cond(carry):
        lo, hi = carry
        return jnp.max(hi - lo) > 1

    def step(carry):
        lo, hi = carry
        mid = lo + ((hi - lo) >> 1)
        ok = count_ge(mid) >= k
        return (jnp.where(ok, mid, lo), jnp.where(ok, hi, mid))

    def body(carry):
        return step(step(carry))

    lo, _ = jax.lax.while_loop(cond, body, (lo0, hi0))
    return lo


def _body(a_ref, o_ref, u_ref, *, n_cols, k):
    x = a_ref[...]
    adj = jnp.tanh(ALPHA_C * x)
    o_ref[...] = adj
    u = jax.lax.bitcast_convert_type(jnp.abs(adj), jnp.int32)
    br = x.shape[0]
    n_pad = u_ref.shape[1]
    u_ref[:, n_cols:n_pad] = jnp.zeros((br, n_pad - n_cols), jnp.int32)
    u_ref[:, 0:n_cols] = u

    w_mx = _LANE
    mx = u_ref[:, 0:w_mx]
    for i in range(1, n_pad // w_mx):
        mx = jnp.maximum(mx, u_ref[:, i * w_mx:(i + 1) * w_mx])
    rowmax = jnp.max(mx, axis=1, keepdims=True)
    rowmin = jnp.min(mx, axis=1, keepdims=True)

    def count_mx(t):
        return jnp.sum((mx >= t).astype(jnp.int32), axis=1, keepdims=True)

    m_k = _bisect_while(count_mx, k, rowmin, rowmax + 1)

    def count_u(t):
        return jnp.sum((u_ref[...] >= t).astype(jnp.int32), axis=1,
                       keepdims=True)

    t_bits = _bisect_while(count_u, k, m_k, rowmax + 1)

    uu = u_ref[...]
    iota = jax.lax.broadcasted_iota(jnp.int32, (br, n_pad), 1)
    u_ref[...] = jnp.where(uu > t_bits, -1,
                           jnp.where(uu == t_bits, iota, _BIG))

    def count_le(t):
        return jnp.sum((u_ref[...] <= t).astype(jnp.int32), axis=1,
                       keepdims=True)

    lo0j = jnp.full((br, 1), -1, jnp.int32)
    hi0j = jnp.full((br, 1), n_cols - 1, jnp.int32)
    clo0 = count_le(lo0j)
    chi0 = count_le(hi0j)

    def jprobe(carry, mid):
        lo, c_lo, hi, c_hi = carry
        c = count_le(mid)
        ok = c >= k
        return (jnp.where(ok, lo, mid), jnp.where(ok, c_lo, c),
                jnp.where(ok, mid, hi), jnp.where(ok, c, c_hi))

    def jcond(carry):
        _, _, _, c_hi = carry
        return jnp.max(c_hi) > k

    def jbody(carry):
        lo, c_lo, hi, c_hi = carry
        span = hi - lo
        frac = ((k - c_lo).astype(jnp.float32) * span.astype(jnp.float32)
                / (c_hi - c_lo).astype(jnp.float32))
        step = jnp.clip(frac.astype(jnp.int32), 1, span - 1)
        carry = jprobe(carry, lo + step)
        lo, c_lo, hi, c_hi = carry
        mid2 = lo + jnp.maximum((hi - lo) >> 1, 1)
        return jprobe(carry, mid2)

    _, _, j_cut, _ = jax.lax.while_loop(jcond, jbody,
                                        (lo0j, clo0, hi0j, chi0))

    keep = u_ref[:, 0:n_cols] <= j_cut
    o_ref[...] = jnp.where(keep, o_ref[...], 0.0)


def kernel(idx, A):
    n, n_cols = A.shape
    del idx
    br = next(b for b in (200, 80, 40, 16, 8, 1) if n % b == 0)
    n_pad = ((n_cols + _LANE - 1) // _LANE) * _LANE
    if n_pad % 256:
        n_pad += _LANE
    body = functools.partial(_body, n_cols=n_cols, k=K_C)
    return pl.pallas_call(
        body,
        grid=(n // br,),
        in_specs=[pl.BlockSpec((br, n_cols), lambda i: (i, 0))],
        out_specs=pl.BlockSpec((br, n_cols), lambda i: (i, 0)),
        out_shape=jax.ShapeDtypeStruct((n, n_cols), jnp.float32),
        scratch_shapes=[
            pltpu.VMEM((br, n_pad), jnp.int32),
        ],
    )(A)
